# Initial kernel scaffold; baseline (speedup 1.0000x reference)
import jax
import jax.numpy as jnp
from jax.experimental import pallas as pl


def kernel(variantxgene_embedding, variantxgene_ixs, W, b, variantxgene_effect):
    raise NotImplementedError("write your pallas kernel here")


# speedup vs baseline: 18.3704x; 18.3704x over previous
import functools

import jax
import jax.numpy as jnp
from jax import lax
from jax.experimental import pallas as pl
from jax.experimental.pallas import tpu as pltpu
from jax.experimental.pallas import tpu_sc as plsc

N_EMB = 16
N_VXG = 1000000
B = 16384
L = 200
N = B * L

NC = 2
NS = 16
NW = NC * NS
N_PER_W = N // NW
CHUNK = 10240
N_CHUNKS = N_PER_W // CHUNK

ROWS = N * N_EMB // 128
BLK = 4096
GRID = ROWS // BLK


def _sc_gather(ixs_hbm, eff_hbm, out_hbm, idx_v, val_v, sem):
    wid = lax.axis_index("s") * NC + lax.axis_index("c")
    base = wid * N_PER_W

    def body(ci, carry):
        cbase = base + ci * CHUNK
        pltpu.sync_copy(ixs_hbm.at[pl.ds(cbase, CHUNK)], idx_v)
        pltpu.async_copy(eff_hbm.at[idx_v], val_v, sem).wait()
        pltpu.sync_copy(val_v, out_hbm.at[pl.ds(cbase, CHUNK)])
        return carry

    lax.fori_loop(0, N_CHUNKS, body, 0)


def _tc_body(x_ref, g_ref, wg_ref, b_ref, o_ref):
    s = lax.dot_general(x_ref[...], wg_ref[...], (((1,), (0,)), ((), ())),
                        preferred_element_type=jnp.float32)
    o_ref[...] = jnp.exp(s + b_ref[...]) * g_ref[...]


def kernel(variantxgene_embedding, variantxgene_ixs, W, b, variantxgene_effect):
    ixs_flat = variantxgene_ixs.reshape(N).astype(jnp.int32)

    mesh = plsc.VectorSubcoreMesh(core_axis_name="c", subcore_axis_name="s")
    gather_run = functools.partial(
        pl.kernel,
        mesh=mesh,
        out_type=jax.ShapeDtypeStruct((N,), jnp.float32),
        scratch_types=[
            pltpu.VMEM((CHUNK,), jnp.int32),
            pltpu.VMEM((CHUNK,), jnp.float32),
            pltpu.SemaphoreType.DMA,
        ],
    )(_sc_gather)
    gathered = gather_run(ixs_flat, variantxgene_effect)

    x2d = variantxgene_embedding.reshape(ROWS, 128)
    g8 = gathered.reshape(ROWS, 8)
    wg = jnp.kron(jnp.eye(8, dtype=jnp.float32), W.reshape(N_EMB, 1))
    b8 = jnp.broadcast_to(b.reshape(1, 1), (1, 8))

    out8 = pl.pallas_call(
        _tc_body,
        grid=(GRID,),
        in_specs=[
            pl.BlockSpec((BLK, 128), lambda i: (i, 0)),
            pl.BlockSpec((BLK, 8), lambda i: (i, 0)),
            pl.BlockSpec((128, 8), lambda i: (0, 0)),
            pl.BlockSpec((1, 8), lambda i: (0, 0)),
        ],
        out_specs=pl.BlockSpec((BLK, 8), lambda i: (i, 0)),
        out_shape=jax.ShapeDtypeStruct((ROWS, 8), jnp.float32),
    )(x2d, g8, wg, b8)
    return out8.reshape(B, L)


# speedup vs baseline: 115.2594x; 6.2742x over previous
import functools

import jax
import jax.numpy as jnp
from jax import lax
from jax.experimental import pallas as pl
from jax.experimental.pallas import tpu as pltpu
from jax.experimental.pallas import tpu_sc as plsc

N_EMB = 16
N_VXG = 1000000
B = 16384
L = 200
N = B * L

NC = 2
NS = 16
NW = NC * NS
N_PER_W = N // NW
CHUNK = 10240
N_CHUNKS = N_PER_W // CHUNK

L_BLK = 8
GRID = L // L_BLK


def _sc_gather(ixs_hbm, eff_hbm, out_hbm, idx_v, val_v, sem):
    wid = lax.axis_index("s") * NC + lax.axis_index("c")
    base = wid * N_PER_W

    def body(ci, carry):
        cbase = base + ci * CHUNK
        pltpu.sync_copy(ixs_hbm.at[pl.ds(cbase, CHUNK)], idx_v)
        pltpu.async_copy(eff_hbm.at[idx_v], val_v, sem).wait()
        pltpu.sync_copy(val_v, out_hbm.at[pl.ds(cbase, CHUNK)])
        return carry

    lax.fori_loop(0, N_CHUNKS, body, 0)


def _tc_body(x_ref, g_ref, w_ref, b_ref, o_ref):
    t = x_ref[...] * w_ref[...]
    s = jnp.sum(t, axis=1) + b_ref[...]
    o_ref[...] = jnp.exp(s) * g_ref[...]


def kernel(variantxgene_embedding, variantxgene_ixs, W, b, variantxgene_effect):
    ixs_t = variantxgene_ixs.T.reshape(N).astype(jnp.int32)
    x_t = jnp.transpose(variantxgene_embedding, (1, 2, 0))

    mesh = plsc.VectorSubcoreMesh(core_axis_name="c", subcore_axis_name="s")
    gather_run = functools.partial(
        pl.kernel,
        mesh=mesh,
        out_type=jax.ShapeDtypeStruct((N,), jnp.float32),
        scratch_types=[
            pltpu.VMEM((CHUNK,), jnp.int32),
            pltpu.VMEM((CHUNK,), jnp.float32),
            pltpu.SemaphoreType.DMA,
        ],
    )(_sc_gather)
    gathered = gather_run(ixs_t, variantxgene_effect)
    g_t = gathered.reshape(L, B)

    out_t = pl.pallas_call(
        _tc_body,
        grid=(GRID,),
        in_specs=[
            pl.BlockSpec((L_BLK, N_EMB, B), lambda i: (i, 0, 0)),
            pl.BlockSpec((L_BLK, B), lambda i: (i, 0)),
            pl.BlockSpec((1, N_EMB, 1), lambda i: (0, 0, 0)),
            pl.BlockSpec((1, 1), lambda i: (0, 0)),
        ],
        out_specs=pl.BlockSpec((L_BLK, B), lambda i: (i, 0)),
        out_shape=jax.ShapeDtypeStruct((L, B), jnp.float32),
    )(x_t, g_t, W.reshape(1, N_EMB, 1), b.reshape(1, 1))
    return out_t.T


# speedup vs baseline: 181.9909x; 1.5790x over previous
import functools

import jax
import jax.numpy as jnp
from jax import lax
from jax.experimental import pallas as pl
from jax.experimental.pallas import tpu as pltpu
from jax.experimental.pallas import tpu_sc as plsc

N_EMB = 16
N_VXG = 1000000
B = 16384
L = 200
N = B * L

K_PHASES = 2
N_PH = N // K_PHASES
L_PH = L // K_PHASES

NC = 2
NS = 16
NW = NC * NS
N_PER_W = N_PH // NW
CHUNK = 12800
N_CHUNKS = N_PER_W // CHUNK

B_BLK = 2048
GRID = B // B_BLK


def _sc_gather(phase, ixs_hbm, eff_hbm, out_hbm,
               idx_v0, idx_v1, val_v0, val_v1, eff_sh,
               sem_i0, sem_i1, sem_g, sem_o0, sem_o1):
    sid = lax.axis_index("s")
    wid = sid * NC + lax.axis_index("c")
    base = phase * N_PH + wid * N_PER_W
    obase = wid * N_PER_W

    @pl.when(sid == 0)
    def _():
        pltpu.sync_copy(eff_hbm, eff_sh)

    idx = [idx_v0, idx_v1]
    val = [val_v0, val_v1]
    sem_i = [sem_i0, sem_i1]
    sem_o = [sem_o0, sem_o1]

    pltpu.async_copy(ixs_hbm.at[pl.ds(base, CHUNK)], idx_v0, sem_i0)
    plsc.subcore_barrier()

    for ci in range(N_CHUNKS):
        cur = ci % 2
        nxt = 1 - cur
        if ci + 1 < N_CHUNKS:
            pltpu.async_copy(
                ixs_hbm.at[pl.ds(base + (ci + 1) * CHUNK, CHUNK)],
                idx[nxt], sem_i[nxt])
        pltpu.make_async_copy(
            ixs_hbm.at[pl.ds(base + ci * CHUNK, CHUNK)],
            idx[cur], sem_i[cur]).wait()
        if ci >= 2:
            pltpu.make_async_copy(
                val[cur], out_hbm.at[pl.ds(obase + (ci - 2) * CHUNK, CHUNK)],
                sem_o[cur]).wait()
        pltpu.async_copy(eff_sh.at[idx[cur]], val[cur], sem_g).wait()
        pltpu.async_copy(
            val[cur], out_hbm.at[pl.ds(obase + ci * CHUNK, CHUNK)],
            sem_o[cur])

    for ci in range(max(N_CHUNKS - 2, 0), N_CHUNKS):
        cur = ci % 2
        pltpu.make_async_copy(
            val[cur], out_hbm.at[pl.ds(obase + ci * CHUNK, CHUNK)],
            sem_o[cur]).wait()


def _tc_body(x_ref, g_ref, w_ref, b_ref, o_ref):
    t = x_ref[...] * w_ref[...]
    s = jnp.sum(t, axis=1) + b_ref[...]
    o_ref[...] = jnp.exp(s) * g_ref[...]


def kernel(variantxgene_embedding, variantxgene_ixs, W, b, variantxgene_effect):
    ixs_t = variantxgene_ixs.T.reshape(N).astype(jnp.int32)
    x_t = jnp.transpose(variantxgene_embedding, (1, 2, 0))

    mesh = plsc.VectorSubcoreMesh(core_axis_name="c", subcore_axis_name="s")
    outs = []
    for k in range(K_PHASES):
        gather_run = functools.partial(
            pl.kernel,
            mesh=mesh,
            out_type=jax.ShapeDtypeStruct((N_PH,), jnp.float32),
            scratch_types=[
                pltpu.VMEM((CHUNK,), jnp.int32),
                pltpu.VMEM((CHUNK,), jnp.int32),
                pltpu.VMEM((CHUNK,), jnp.float32),
                pltpu.VMEM((CHUNK,), jnp.float32),
                pltpu.VMEM_SHARED((N_VXG,), jnp.float32),
                pltpu.SemaphoreType.DMA,
                pltpu.SemaphoreType.DMA,
                pltpu.SemaphoreType.DMA,
                pltpu.SemaphoreType.DMA,
                pltpu.SemaphoreType.DMA,
            ],
        )(functools.partial(_sc_gather, k))
        gathered = gather_run(ixs_t, variantxgene_effect)
        g_t = gathered.reshape(L_PH, B)

        out_k = pl.pallas_call(
            _tc_body,
            grid=(GRID,),
            in_specs=[
                pl.BlockSpec((L_PH, N_EMB, B_BLK),
                             functools.partial(lambda k, i: (k, 0, i), k)),
                pl.BlockSpec((L_PH, B_BLK), lambda i: (0, i)),
                pl.BlockSpec((1, N_EMB, 1), lambda i: (0, 0, 0)),
                pl.BlockSpec((1, 1), lambda i: (0, 0)),
            ],
            out_specs=pl.BlockSpec((L_PH, B_BLK), lambda i: (0, i)),
            out_shape=jax.ShapeDtypeStruct((L_PH, B), jnp.float32),
        )(x_t, g_t, W.reshape(1, N_EMB, 1), b.reshape(1, 1))
        outs.append(out_k)

    return jnp.concatenate(outs, axis=0).T


# speedup vs baseline: 199.8198x; 1.0980x over previous
import functools

import jax
import jax.numpy as jnp
from jax import lax
from jax.experimental import pallas as pl
from jax.experimental.pallas import tpu as pltpu
from jax.experimental.pallas import tpu_sc as plsc

N_EMB = 16
N_VXG = 1000000
B = 16384
L = 200
N = B * L

K_PHASES = 5
L_PH = L // K_PHASES
N_PH = L_PH * B

NC = 2
NS = 16
NW = NC * NS
N_PER_W = N_PH // NW
CHUNK = 10240
N_CHUNKS = N_PER_W // CHUNK

B_BLK = 2048
GRID = B // B_BLK


def _sc_gather(phase, ixs_hbm, eff_hbm, out_hbm,
               idx_v0, idx_v1, val_v0, val_v1, eff_sh,
               sem_i0, sem_i1, sem_g, sem_o0, sem_o1):
    sid = lax.axis_index("s")
    wid = sid * NC + lax.axis_index("c")
    base = phase * N_PH + wid * N_PER_W
    obase = wid * N_PER_W

    @pl.when(sid == 0)
    def _():
        pltpu.sync_copy(eff_hbm, eff_sh)

    idx = [idx_v0, idx_v1]
    val = [val_v0, val_v1]
    sem_i = [sem_i0, sem_i1]
    sem_o = [sem_o0, sem_o1]

    pltpu.async_copy(ixs_hbm.at[pl.ds(base, CHUNK)], idx_v0, sem_i0)
    plsc.subcore_barrier()

    for ci in range(N_CHUNKS):
        cur = ci % 2
        nxt = 1 - cur
        if ci + 1 < N_CHUNKS:
            pltpu.async_copy(
                ixs_hbm.at[pl.ds(base + (ci + 1) * CHUNK, CHUNK)],
                idx[nxt], sem_i[nxt])
        pltpu.make_async_copy(
            ixs_hbm.at[pl.ds(base + ci * CHUNK, CHUNK)],
            idx[cur], sem_i[cur]).wait()
        if ci >= 2:
            pltpu.make_async_copy(
                val[cur], out_hbm.at[pl.ds(obase + (ci - 2) * CHUNK, CHUNK)],
                sem_o[cur]).wait()
        pltpu.async_copy(eff_sh.at[idx[cur]], val[cur], sem_g).wait()
        pltpu.async_copy(
            val[cur], out_hbm.at[pl.ds(obase + ci * CHUNK, CHUNK)],
            sem_o[cur])

    for ci in range(max(N_CHUNKS - 2, 0), N_CHUNKS):
        cur = ci % 2
        pltpu.make_async_copy(
            val[cur], out_hbm.at[pl.ds(obase + ci * CHUNK, CHUNK)],
            sem_o[cur]).wait()


def _tc_body(x_ref, g_ref, w_ref, b_ref, o_ref):
    t = x_ref[...] * w_ref[...]
    s = jnp.sum(t, axis=1) + b_ref[...]
    o_ref[...] = jnp.exp(s) * g_ref[...]


def kernel(variantxgene_embedding, variantxgene_ixs, W, b, variantxgene_effect):
    ixs_raw = (variantxgene_ixs.T.reshape(L // 8, 8, B // 128, 128)
               .transpose(0, 2, 1, 3).reshape(N).astype(jnp.int32))
    x_t = jnp.transpose(variantxgene_embedding, (1, 2, 0))

    mesh = plsc.VectorSubcoreMesh(core_axis_name="c", subcore_axis_name="s")
    outs = []
    for k in range(K_PHASES):
        gather_run = functools.partial(
            pl.kernel,
            mesh=mesh,
            out_type=jax.ShapeDtypeStruct((N_PH,), jnp.float32),
            scratch_types=[
                pltpu.VMEM((CHUNK,), jnp.int32),
                pltpu.VMEM((CHUNK,), jnp.int32),
                pltpu.VMEM((CHUNK,), jnp.float32),
                pltpu.VMEM((CHUNK,), jnp.float32),
                pltpu.VMEM_SHARED((N_VXG,), jnp.float32),
                pltpu.SemaphoreType.DMA,
                pltpu.SemaphoreType.DMA,
                pltpu.SemaphoreType.DMA,
                pltpu.SemaphoreType.DMA,
                pltpu.SemaphoreType.DMA,
            ],
        )(functools.partial(_sc_gather, k))
        gathered = gather_run(ixs_raw, variantxgene_effect)
        g_t = (gathered.reshape(L_PH // 8, B // 128, 8, 128)
               .transpose(0, 2, 1, 3).reshape(L_PH, B))

        out_k = pl.pallas_call(
            _tc_body,
            grid=(GRID,),
            in_specs=[
                pl.BlockSpec((L_PH, N_EMB, B_BLK),
                             functools.partial(lambda k, i: (k, 0, i), k)),
                pl.BlockSpec((L_PH, B_BLK), lambda i: (0, i)),
                pl.BlockSpec((1, N_EMB, 1), lambda i: (0, 0, 0)),
                pl.BlockSpec((1, 1), lambda i: (0, 0)),
            ],
            out_specs=pl.BlockSpec((L_PH, B_BLK), lambda i: (0, i)),
            out_shape=jax.ShapeDtypeStruct((L_PH, B), jnp.float32),
        )(x_t, g_t, W.reshape(1, N_EMB, 1), b.reshape(1, 1))
        outs.append(out_k)

    return jnp.concatenate(outs, axis=0).T


# speedup vs baseline: 219.0322x; 1.0961x over previous
import functools

import jax
import jax.numpy as jnp
from jax import lax
from jax.experimental import pallas as pl
from jax.experimental.pallas import tpu as pltpu
from jax.experimental.pallas import tpu_sc as plsc

N_EMB = 16
N_VXG = 1000000
B = 16384
L = 200
N = B * L

K_PHASES = 5
L_PH = L // K_PHASES
N_PH = L_PH * B

NC = 2
NS = 16
NW = NC * NS
N_PER_W = N_PH // NW
CHUNK = 10240
N_CHUNKS = N_PER_W // CHUNK

B_BLK = 4096
GRID = B // B_BLK


def _sc_gather(phase, ixs_hbm, eff_hbm, out_hbm,
               idx_v0, idx_v1, val_v0, val_v1, eff_sh,
               sem_i0, sem_i1, sem_g, sem_o0, sem_o1):
    sid = lax.axis_index("s")
    wid = sid * NC + lax.axis_index("c")
    base = phase * N_PH + wid * N_PER_W
    obase = wid * N_PER_W

    @pl.when(sid == 0)
    def _():
        pltpu.sync_copy(eff_hbm, eff_sh)

    idx = [idx_v0, idx_v1]
    val = [val_v0, val_v1]
    sem_i = [sem_i0, sem_i1]
    sem_o = [sem_o0, sem_o1]

    pltpu.async_copy(ixs_hbm.at[pl.ds(base, CHUNK)], idx_v0, sem_i0)
    plsc.subcore_barrier()

    for ci in range(N_CHUNKS):
        cur = ci % 2
        nxt = 1 - cur
        if ci + 1 < N_CHUNKS:
            pltpu.async_copy(
                ixs_hbm.at[pl.ds(base + (ci + 1) * CHUNK, CHUNK)],
                idx[nxt], sem_i[nxt])
        pltpu.make_async_copy(
            ixs_hbm.at[pl.ds(base + ci * CHUNK, CHUNK)],
            idx[cur], sem_i[cur]).wait()
        if ci >= 2:
            pltpu.make_async_copy(
                val[cur], out_hbm.at[pl.ds(obase + (ci - 2) * CHUNK, CHUNK)],
                sem_o[cur]).wait()
        pltpu.async_copy(eff_sh.at[idx[cur]], val[cur], sem_g).wait()
        pltpu.async_copy(
            val[cur], out_hbm.at[pl.ds(obase + ci * CHUNK, CHUNK)],
            sem_o[cur])

    for ci in range(max(N_CHUNKS - 2, 0), N_CHUNKS):
        cur = ci % 2
        pltpu.make_async_copy(
            val[cur], out_hbm.at[pl.ds(obase + ci * CHUNK, CHUNK)],
            sem_o[cur]).wait()


def _tc_body(x_ref, g_ref, w_ref, b_ref, acc_ref, o_ref):
    del acc_ref
    t = x_ref[...] * w_ref[...]
    s = jnp.sum(t, axis=1) + b_ref[...]
    o_ref[...] = jnp.exp(s) * g_ref[...]


def kernel(variantxgene_embedding, variantxgene_ixs, W, b, variantxgene_effect):
    ixs_raw = (variantxgene_ixs.T.reshape(L // 8, 8, B // 128, 128)
               .transpose(0, 2, 1, 3).reshape(N).astype(jnp.int32))
    x_t = jnp.transpose(variantxgene_embedding, (1, 2, 0))

    mesh = plsc.VectorSubcoreMesh(core_axis_name="c", subcore_axis_name="s")
    acc = jnp.zeros((L, B), jnp.float32)
    for k in range(K_PHASES):
        gather_run = functools.partial(
            pl.kernel,
            mesh=mesh,
            out_type=jax.ShapeDtypeStruct((N_PH,), jnp.float32),
            scratch_types=[
                pltpu.VMEM((CHUNK,), jnp.int32),
                pltpu.VMEM((CHUNK,), jnp.int32),
                pltpu.VMEM((CHUNK,), jnp.float32),
                pltpu.VMEM((CHUNK,), jnp.float32),
                pltpu.VMEM_SHARED((N_VXG,), jnp.float32),
                pltpu.SemaphoreType.DMA,
                pltpu.SemaphoreType.DMA,
                pltpu.SemaphoreType.DMA,
                pltpu.SemaphoreType.DMA,
                pltpu.SemaphoreType.DMA,
            ],
        )(functools.partial(_sc_gather, k))
        gathered = gather_run(ixs_raw, variantxgene_effect)
        g_t = (gathered.reshape(L_PH // 8, B // 128, 8, 128)
               .transpose(0, 2, 1, 3).reshape(L_PH, B))

        acc = pl.pallas_call(
            _tc_body,
            grid=(GRID,),
            in_specs=[
                pl.BlockSpec((L_PH, N_EMB, B_BLK),
                             functools.partial(lambda k, i: (k, 0, i), k)),
                pl.BlockSpec((L_PH, B_BLK), lambda i: (0, i)),
                pl.BlockSpec((1, N_EMB, 1), lambda i: (0, 0, 0)),
                pl.BlockSpec((1, 1), lambda i: (0, 0)),
                pl.BlockSpec(memory_space=pl.ANY),
            ],
            out_specs=pl.BlockSpec(
                (L_PH, B_BLK),
                functools.partial(lambda k, i: (k, i), k)),
            out_shape=jax.ShapeDtypeStruct((L, B), jnp.float32),
            input_output_aliases={4: 0},
        )(x_t, g_t, W.reshape(1, N_EMB, 1), b.reshape(1, 1), acc)

    return acc.T


# speedup vs baseline: 236.5253x; 1.0799x over previous
import functools

import jax
import jax.numpy as jnp
from jax import lax
from jax.experimental import pallas as pl
from jax.experimental.pallas import tpu as pltpu
from jax.experimental.pallas import tpu_sc as plsc

N_EMB = 16
N_VXG = 1000000
B = 16384
L = 200
N = B * L

K_PHASES = 5
L_PH = L // K_PHASES
N_PH = L_PH * B

NC = 2
NS = 16
NW = NC * NS
N_PER_W = N_PH // NW
CHUNK = 10240
N_CHUNKS = N_PER_W // CHUNK

B_BLK = 4096
GRID = B // B_BLK


def _sc_gather(phase, ixs_hbm, eff_hbm, out_hbm,
               idx_v0, idx_v1, val_v0, val_v1, eff_sh,
               sem_i0, sem_i1, sem_g, sem_o0, sem_o1):
    sid = lax.axis_index("s")
    wid = sid * NC + lax.axis_index("c")
    base = phase * N_PH + wid * N_PER_W
    obase = wid * N_PER_W

    @pl.when(sid == 0)
    def _():
        pltpu.sync_copy(eff_hbm, eff_sh)

    idx = [idx_v0, idx_v1]
    val = [val_v0, val_v1]
    sem_i = [sem_i0, sem_i1]
    sem_o = [sem_o0, sem_o1]

    pltpu.async_copy(ixs_hbm.at[pl.ds(base, CHUNK)], idx_v0, sem_i0)
    plsc.subcore_barrier()

    for ci in range(N_CHUNKS):
        cur = ci % 2
        nxt = 1 - cur
        if ci + 1 < N_CHUNKS:
            pltpu.async_copy(
                ixs_hbm.at[pl.ds(base + (ci + 1) * CHUNK, CHUNK)],
                idx[nxt], sem_i[nxt])
        pltpu.make_async_copy(
            ixs_hbm.at[pl.ds(base + ci * CHUNK, CHUNK)],
            idx[cur], sem_i[cur]).wait()
        if ci >= 2:
            pltpu.make_async_copy(
                val[cur], out_hbm.at[pl.ds(obase + (ci - 2) * CHUNK, CHUNK)],
                sem_o[cur]).wait()
        pltpu.async_copy(eff_sh.at[idx[cur]], val[cur], sem_g).wait()
        pltpu.async_copy(
            val[cur], out_hbm.at[pl.ds(obase + ci * CHUNK, CHUNK)],
            sem_o[cur])

    for ci in range(max(N_CHUNKS - 2, 0), N_CHUNKS):
        cur = ci % 2
        pltpu.make_async_copy(
            val[cur], out_hbm.at[pl.ds(obase + ci * CHUNK, CHUNK)],
            sem_o[cur]).wait()


def _tc_body(x_ref, g_ref, w_ref, b_ref, acc_ref, o_ref):
    del acc_ref
    acc = x_ref[:, 0, :] * w_ref[0]
    for k in range(1, N_EMB):
        acc = acc + x_ref[:, k, :] * w_ref[k]
    o_ref[...] = jnp.exp(acc + b_ref[0]) * g_ref[...]


def kernel(variantxgene_embedding, variantxgene_ixs, W, b, variantxgene_effect):
    ixs_raw = (variantxgene_ixs.T.reshape(L // 8, 8, B // 128, 128)
               .transpose(0, 2, 1, 3).reshape(N).astype(jnp.int32))
    x_t = jnp.transpose(variantxgene_embedding, (1, 2, 0))

    mesh = plsc.VectorSubcoreMesh(core_axis_name="c", subcore_axis_name="s")
    acc = jnp.zeros((L, B), jnp.float32)
    for k in range(K_PHASES):
        gather_run = functools.partial(
            pl.kernel,
            mesh=mesh,
            out_type=jax.ShapeDtypeStruct((N_PH,), jnp.float32),
            scratch_types=[
                pltpu.VMEM((CHUNK,), jnp.int32),
                pltpu.VMEM((CHUNK,), jnp.int32),
                pltpu.VMEM((CHUNK,), jnp.float32),
                pltpu.VMEM((CHUNK,), jnp.float32),
                pltpu.VMEM_SHARED((N_VXG,), jnp.float32),
                pltpu.SemaphoreType.DMA,
                pltpu.SemaphoreType.DMA,
                pltpu.SemaphoreType.DMA,
                pltpu.SemaphoreType.DMA,
                pltpu.SemaphoreType.DMA,
            ],
        )(functools.partial(_sc_gather, k))
        gathered = gather_run(ixs_raw, variantxgene_effect)
        g_t = (gathered.reshape(L_PH // 8, B // 128, 8, 128)
               .transpose(0, 2, 1, 3).reshape(L_PH, B))

        acc = pl.pallas_call(
            _tc_body,
            grid=(GRID,),
            in_specs=[
                pl.BlockSpec((L_PH, N_EMB, B_BLK),
                             functools.partial(lambda k, i: (k, 0, i), k)),
                pl.BlockSpec((L_PH, B_BLK), lambda i: (0, i)),
                pl.BlockSpec(memory_space=pltpu.SMEM),
                pl.BlockSpec(memory_space=pltpu.SMEM),
                pl.BlockSpec(memory_space=pl.ANY),
            ],
            out_specs=pl.BlockSpec(
                (L_PH, B_BLK),
                functools.partial(lambda k, i: (k, i), k)),
            out_shape=jax.ShapeDtypeStruct((L, B), jnp.float32),
            input_output_aliases={4: 0},
        )(x_t, g_t, W.reshape(N_EMB), b, acc)

    return acc.T


# speedup vs baseline: 238.7059x; 1.0092x over previous
import functools

import jax
import jax.numpy as jnp
from jax import lax
from jax.experimental import pallas as pl
from jax.experimental.pallas import tpu as pltpu
from jax.experimental.pallas import tpu_sc as plsc

N_EMB = 16
N_VXG = 1000000
B = 16384
L = 200
N = B * L

K_PHASES = 5
L_PH = L // K_PHASES
N_PH = L_PH * B

NC = 2
NS = 16
NW = NC * NS
N_PER_W = N_PH // NW
CHUNK = 10240
N_CHUNKS = N_PER_W // CHUNK

B_BLK = 4096
GRID = B // B_BLK


def _sc_gather(phase, ixs_hbm, eff_hbm, out_hbm,
               idx_v0, idx_v1, val_v0, val_v1, eff_sh,
               sem_i0, sem_i1, sem_g, sem_o0, sem_o1):
    sid = lax.axis_index("s")
    wid = sid * NC + lax.axis_index("c")
    base = phase * N_PH + wid * N_PER_W
    obase = wid * N_PER_W

    @pl.when(sid == 0)
    def _():
        pltpu.sync_copy(eff_hbm, eff_sh)

    idx = [idx_v0, idx_v1]
    val = [val_v0, val_v1]
    sem_i = [sem_i0, sem_i1]
    sem_o = [sem_o0, sem_o1]

    pltpu.async_copy(ixs_hbm.at[pl.ds(base, CHUNK)], idx_v0, sem_i0)
    plsc.subcore_barrier()

    for ci in range(N_CHUNKS):
        cur = ci % 2
        nxt = 1 - cur
        if ci + 1 < N_CHUNKS:
            pltpu.async_copy(
                ixs_hbm.at[pl.ds(base + (ci + 1) * CHUNK, CHUNK)],
                idx[nxt], sem_i[nxt])
        pltpu.make_async_copy(
            ixs_hbm.at[pl.ds(base + ci * CHUNK, CHUNK)],
            idx[cur], sem_i[cur]).wait()
        if ci >= 2:
            pltpu.make_async_copy(
                val[cur], out_hbm.at[pl.ds(obase + (ci - 2) * CHUNK, CHUNK)],
                sem_o[cur]).wait()
        pltpu.async_copy(eff_sh.at[idx[cur]], val[cur], sem_g).wait()
        pltpu.async_copy(
            val[cur], out_hbm.at[pl.ds(obase + ci * CHUNK, CHUNK)],
            sem_o[cur])

    for ci in range(max(N_CHUNKS - 2, 0), N_CHUNKS):
        cur = ci % 2
        pltpu.make_async_copy(
            val[cur], out_hbm.at[pl.ds(obase + ci * CHUNK, CHUNK)],
            sem_o[cur]).wait()


def _tc_body(x_ref, g_ref, w_ref, b_ref, *rest):
    o_ref = rest[-1]
    acc = x_ref[:, 0, :] * w_ref[0]
    for k in range(1, N_EMB):
        acc = acc + x_ref[:, k, :] * w_ref[k]
    o_ref[...] = jnp.exp(acc + b_ref[0]) * g_ref[...]


def kernel(variantxgene_embedding, variantxgene_ixs, W, b, variantxgene_effect):
    ixs_raw = (variantxgene_ixs.T.reshape(L // 8, 8, B // 128, 128)
               .transpose(0, 2, 1, 3).reshape(N).astype(jnp.int32))
    x_t = jnp.transpose(variantxgene_embedding, (1, 2, 0))

    mesh = plsc.VectorSubcoreMesh(core_axis_name="c", subcore_axis_name="s")
    acc = None
    for k in range(K_PHASES):
        gather_run = functools.partial(
            pl.kernel,
            mesh=mesh,
            out_type=jax.ShapeDtypeStruct((N_PH,), jnp.float32),
            scratch_types=[
                pltpu.VMEM((CHUNK,), jnp.int32),
                pltpu.VMEM((CHUNK,), jnp.int32),
                pltpu.VMEM((CHUNK,), jnp.float32),
                pltpu.VMEM((CHUNK,), jnp.float32),
                pltpu.VMEM_SHARED((N_VXG,), jnp.float32),
                pltpu.SemaphoreType.DMA,
                pltpu.SemaphoreType.DMA,
                pltpu.SemaphoreType.DMA,
                pltpu.SemaphoreType.DMA,
                pltpu.SemaphoreType.DMA,
            ],
        )(functools.partial(_sc_gather, k))
        gathered = gather_run(ixs_raw, variantxgene_effect)
        g_t = (gathered.reshape(L_PH // 8, B // 128, 8, 128)
               .transpose(0, 2, 1, 3).reshape(L_PH, B))

        in_specs = [
            pl.BlockSpec((L_PH, N_EMB, B_BLK),
                         functools.partial(lambda k, i: (k, 0, i), k)),
            pl.BlockSpec((L_PH, B_BLK), lambda i: (0, i)),
            pl.BlockSpec(memory_space=pltpu.SMEM),
            pl.BlockSpec(memory_space=pltpu.SMEM),
        ]
        args = [x_t, g_t, W.reshape(N_EMB), b]
        aliases = {}
        if acc is not None:
            in_specs.append(pl.BlockSpec(memory_space=pl.ANY))
            args.append(acc)
            aliases = {4: 0}
        acc = pl.pallas_call(
            _tc_body,
            grid=(GRID,),
            in_specs=in_specs,
            out_specs=pl.BlockSpec(
                (L_PH, B_BLK),
                functools.partial(lambda k, i: (k, i), k)),
            out_shape=jax.ShapeDtypeStruct((L, B), jnp.float32),
            input_output_aliases=aliases,
        )(*args)

    return acc.T
